# Initial kernel scaffold; baseline (speedup 1.0000x reference)
import jax
import jax.numpy as jnp
from jax.experimental import pallas as pl


def kernel(prob, edge_index, w_p, w_n):
    raise NotImplementedError("write your pallas kernel here")


# speedup vs baseline: 24.2565x; 24.2565x over previous
import functools

import jax
import jax.numpy as jnp
from jax import lax
from jax.experimental import pallas as pl
from jax.experimental.pallas import tpu as pltpu
from jax.experimental.pallas import tpu_sc as plsc

L = 16
NC = 2
NS = 16
NW = NC * NS
CH = 1024
GW = 128


def _sc_edge_kernel(n_pad, e_pad, nchunk, rpt):
    mesh = plsc.VectorSubcoreMesh(core_axis_name="c", subcore_axis_name="s")

    @functools.partial(
        pl.kernel,
        mesh=mesh,
        compiler_params=pltpu.CompilerParams(use_tc_tiling_on_sc=False),
        out_type=(
            jax.ShapeDtypeStruct((NC, L), jnp.float32),
            jax.ShapeDtypeStruct((NC, L), jnp.float32),
        ),
        scratch_types=[
            pltpu.VMEM((CH // GW, GW), jnp.int32),
            pltpu.VMEM((CH // GW, GW), jnp.int32),
            pltpu.VMEM((CH,), jnp.float32),
            pltpu.VMEM((CH,), jnp.float32),
            pltpu.VMEM((CH, L), jnp.float32),
            pltpu.VMEM((CH, L), jnp.float32),
            pltpu.VMEM((rpt, L), jnp.float32),
            pltpu.VMEM((L,), jnp.float32),
            pltpu.VMEM((L,), jnp.float32),
            pltpu.VMEM_SHARED((NS, L), jnp.float32),
            pltpu.VMEM_SHARED((NS, L), jnp.float32),
            pltpu.SemaphoreType.DMA,
        ],
    )
    def k(prob_hbm, row_hbm, col_hbm, wp_hbm, wn_hbm, num_out, den_out,
          idx_a, idx_b, wp_v, wn_v, rows_a, rows_b, probs_v, acc_v, tmp_v,
          shared_num, shared_den, sem):
        c = lax.axis_index("c")
        s = lax.axis_index("s")
        wid = s * NC + c
        per = nchunk * CH
        ebase = wid * per
        ebase_g = ebase // GW

        def chunk_body(g, num_acc):
            goff = pl.multiple_of(ebase_g + g * (CH // GW), 8)
            eoff = pl.multiple_of(ebase + g * CH, CH)
            pltpu.sync_copy(row_hbm.at[pl.ds(goff, CH // GW)], idx_a)
            pltpu.sync_copy(col_hbm.at[pl.ds(goff, CH // GW)], idx_b)
            pltpu.sync_copy(wp_hbm.at[pl.ds(eoff, CH)], wp_v)
            pltpu.sync_copy(wn_hbm.at[pl.ds(eoff, CH)], wn_v)
            cps = []
            for j in range(CH // GW):
                cps.append(pltpu.async_copy(
                    prob_hbm.at[idx_a.at[j]], rows_a.at[pl.ds(j * GW, GW)], sem))
            for j in range(CH // GW):
                cps.append(pltpu.async_copy(
                    prob_hbm.at[idx_b.at[j]], rows_b.at[pl.ds(j * GW, GW)], sem))
            for cp in cps:
                cp.wait()

            def group_body(gi, acc):
                eb = gi * L
                wp16 = wp_v[pl.ds(eb, L)]
                wn16 = wn_v[pl.ds(eb, L)]
                for j in range(L):
                    a = rows_a[eb + j]
                    b = rows_b[eb + j]
                    wpe = jnp.full((L,), wp16[j], jnp.float32)
                    wne = jnp.full((L,), wn16[j], jnp.float32)
                    acc = acc + a * (wpe * a + (wne - wpe) * b)
                return acc

            return lax.fori_loop(0, CH // L, group_body, num_acc)

        num_acc = lax.fori_loop(0, nchunk, chunk_body,
                                jnp.zeros((L,), jnp.float32))

        pltpu.sync_copy(prob_hbm.at[pl.ds(pl.multiple_of(wid * rpt, 8), rpt)],
                        probs_v)

        def den_body(i, acc):
            r = probs_v[i]
            return acc + r * r

        den_acc = lax.fori_loop(0, rpt, den_body,
                                jnp.zeros((L,), jnp.float32), unroll=4)

        acc_v[...] = num_acc
        pltpu.sync_copy(acc_v, shared_num.at[s])
        acc_v[...] = den_acc
        pltpu.sync_copy(acc_v, shared_den.at[s])
        plsc.subcore_barrier()

        @pl.when(s == 0)
        def _():
            num_tot = jnp.zeros((L,), jnp.float32)
            den_tot = jnp.zeros((L,), jnp.float32)
            for j in range(NS):
                pltpu.sync_copy(shared_num.at[j], tmp_v)
                num_tot = num_tot + tmp_v[...]
                pltpu.sync_copy(shared_den.at[j], tmp_v)
                den_tot = den_tot + tmp_v[...]
            acc_v[...] = num_tot
            pltpu.sync_copy(acc_v, num_out.at[c])
            acc_v[...] = den_tot
            pltpu.sync_copy(acc_v, den_out.at[c])

    return k


def _combine_body(num_ref, den_ref, out_ref):
    num = jnp.sum(num_ref[...], axis=0, keepdims=True)
    den = jnp.sum(den_ref[...], axis=0, keepdims=True) + 1.0
    out_ref[...] = jnp.full((1, L), jnp.sum(num / den), jnp.float32)


def kernel(prob, edge_index, w_p, w_n):
    n, kk = prob.shape
    e = w_p.shape[0]

    rpt = ((-(-n // NW)) + 7) // 8 * 8
    n_pad = rpt * NW
    prob_pad = jnp.zeros((n_pad, L), jnp.float32).at[:n, :kk].set(prob)

    nchunk = -(-e // (NW * CH))
    e_pad = nchunk * CH * NW
    pad = e_pad - e
    row = jnp.concatenate([edge_index[0], jnp.zeros((pad,), jnp.int32)])
    col = jnp.concatenate([edge_index[1], jnp.zeros((pad,), jnp.int32)])
    wp = jnp.concatenate([w_p, jnp.zeros((pad,), jnp.float32)])
    wn = jnp.concatenate([w_n, jnp.zeros((pad,), jnp.float32)])
    row2 = row.reshape(e_pad // GW, GW)
    col2 = col.reshape(e_pad // GW, GW)

    num_parts, den_parts = _sc_edge_kernel(n_pad, e_pad, nchunk, rpt)(
        prob_pad, row2, col2, wp, wn)

    out = pl.pallas_call(
        _combine_body,
        out_shape=jax.ShapeDtypeStruct((1, L), jnp.float32),
    )(num_parts, den_parts)
    return out[0, 0:1]


# speedup vs baseline: 44.8328x; 1.8483x over previous
import functools

import jax
import jax.numpy as jnp
from jax import lax
from jax.experimental import pallas as pl
from jax.experimental.pallas import tpu as pltpu
from jax.experimental.pallas import tpu_sc as plsc

L = 16
NC = 2
NS = 16
NW = NC * NS
CH = 400
GW = 80
NG = CH // GW


def _sc_edge_kernel(nchunk, rpt):
    mesh = plsc.VectorSubcoreMesh(core_axis_name="c", subcore_axis_name="s")
    per = nchunk * CH

    @functools.partial(
        pl.kernel,
        mesh=mesh,
        compiler_params=pltpu.CompilerParams(use_tc_tiling_on_sc=False),
        out_type=(
            jax.ShapeDtypeStruct((NC, L), jnp.float32),
            jax.ShapeDtypeStruct((NC, L), jnp.float32),
        ),
        scratch_types=[
            pltpu.VMEM((4, CH), jnp.int32),
            pltpu.VMEM((4, CH), jnp.int32),
            pltpu.VMEM((4, CH), jnp.float32),
            pltpu.VMEM((4, CH), jnp.float32),
            pltpu.VMEM((2, CH, L), jnp.float32),
            pltpu.VMEM((2, CH, L), jnp.float32),
            pltpu.VMEM((rpt, L), jnp.float32),
            pltpu.VMEM((L,), jnp.float32),
            pltpu.VMEM((L,), jnp.float32),
            pltpu.VMEM_SHARED((NS, L), jnp.float32),
            pltpu.VMEM_SHARED((NS, L), jnp.float32),
            pltpu.SemaphoreType.DMA,
            pltpu.SemaphoreType.DMA,
            pltpu.SemaphoreType.DMA,
            pltpu.SemaphoreType.DMA,
            pltpu.SemaphoreType.DMA,
            pltpu.SemaphoreType.DMA,
            pltpu.SemaphoreType.DMA,
        ],
    )
    def k(prob_hbm, row_hbm, col_hbm, wp_hbm, wn_hbm, num_out, den_out,
          idx_a, idx_b, wp_v, wn_v, rows_a, rows_b, probs_v, acc_v, tmp_v,
          shared_num, shared_den,
          sem_l0, sem_l1, sem_l2, sem_l3, sem_g0, sem_g1, sem_p):
        c = lax.axis_index("c")
        s = lax.axis_index("s")
        wid = s * NC + c
        ebase = wid * per
        sem_l = (sem_l0, sem_l1, sem_l2, sem_l3)
        sem_g = (sem_g0, sem_g1)

        pcp = pltpu.async_copy(
            prob_hbm.at[pl.ds(pl.multiple_of(wid * rpt, 8), rpt)],
            probs_v, sem_p)

        def fire_lin(g, b):
            off = pl.multiple_of(ebase + g * CH, 8)
            pltpu.async_copy(row_hbm.at[pl.ds(off, CH)], idx_a.at[b], sem_l[b])
            pltpu.async_copy(col_hbm.at[pl.ds(off, CH)], idx_b.at[b], sem_l[b])
            pltpu.async_copy(wp_hbm.at[pl.ds(off, CH)], wp_v.at[b], sem_l[b])
            pltpu.async_copy(wn_hbm.at[pl.ds(off, CH)], wn_v.at[b], sem_l[b])

        def wait_lin(b):
            pltpu.make_async_copy(row_hbm.at[pl.ds(0, CH)], idx_a.at[b],
                                  sem_l[b]).wait()
            pltpu.make_async_copy(col_hbm.at[pl.ds(0, CH)], idx_b.at[b],
                                  sem_l[b]).wait()
            pltpu.make_async_copy(wp_hbm.at[pl.ds(0, CH)], wp_v.at[b],
                                  sem_l[b]).wait()
            pltpu.make_async_copy(wn_hbm.at[pl.ds(0, CH)], wn_v.at[b],
                                  sem_l[b]).wait()

        def fire_gath(lb, rb):
            for j in range(NG):
                sl = pl.ds(j * GW, GW)
                pltpu.async_copy(prob_hbm.at[idx_a.at[lb].at[sl]],
                                 rows_a.at[rb].at[sl], sem_g[rb])
                pltpu.async_copy(prob_hbm.at[idx_b.at[lb].at[sl]],
                                 rows_b.at[rb].at[sl], sem_g[rb])

        def wait_gath(rb):
            for j in range(NG):
                sl = pl.ds(j * GW, GW)
                pltpu.make_async_copy(prob_hbm.at[pl.ds(0, GW)],
                                      rows_a.at[rb].at[sl], sem_g[rb]).wait()
                pltpu.make_async_copy(prob_hbm.at[pl.ds(0, GW)],
                                      rows_b.at[rb].at[sl], sem_g[rb]).wait()

        def compute(lb, rb, acc):
            ra = rows_a.at[rb]
            rbv = rows_b.at[rb]
            wpr = wp_v.at[lb]
            wnr = wn_v.at[lb]

            def group_body(gi, acc):
                eb = gi * L
                wp16 = wpr[pl.ds(eb, L)]
                wd16 = wnr[pl.ds(eb, L)] - wp16
                for j in range(L):
                    a = ra[eb + j]
                    b = rbv[eb + j]
                    wpe = jnp.full((L,), wp16[j], jnp.float32)
                    wde = jnp.full((L,), wd16[j], jnp.float32)
                    acc = acc + a * (wpe * a + wde * b)
                return acc

            return lax.fori_loop(0, CH // L, group_body, acc)

        nq = nchunk // 4

        fire_lin(0, 0)
        fire_lin(1, 1)
        wait_lin(0)
        fire_gath(0, 0)

        def quad_body(i, acc):
            g0 = 4 * i
            wait_lin(1)
            fire_gath(1, 1)
            fire_lin(g0 + 2, 2)
            wait_gath(0)
            acc = compute(0, 0, acc)
            wait_lin(2)
            fire_gath(2, 0)
            fire_lin(g0 + 3, 3)
            wait_gath(1)
            acc = compute(1, 1, acc)
            wait_lin(3)
            fire_gath(3, 1)
            fire_lin(g0 + 4, 0)
            wait_gath(0)
            acc = compute(2, 0, acc)
            wait_lin(0)
            fire_gath(0, 0)
            fire_lin(jnp.minimum(g0 + 5, nchunk - 1), 1)
            wait_gath(1)
            acc = compute(3, 1, acc)
            return acc

        num_acc = lax.fori_loop(0, nq, quad_body, jnp.zeros((L,), jnp.float32))

        wait_gath(0)
        num_acc = compute(0, 0, num_acc)
        wait_lin(1)

        pcp.wait()

        def den_body(i, acc):
            r = probs_v[i]
            return acc + r * r

        den_acc = lax.fori_loop(0, rpt, den_body,
                                jnp.zeros((L,), jnp.float32))

        acc_v[...] = num_acc
        pltpu.sync_copy(acc_v, shared_num.at[s])
        acc_v[...] = den_acc
        pltpu.sync_copy(acc_v, shared_den.at[s])
        plsc.subcore_barrier()

        @pl.when(s == 0)
        def _():
            num_tot = jnp.zeros((L,), jnp.float32)
            den_tot = jnp.zeros((L,), jnp.float32)
            for j in range(NS):
                pltpu.sync_copy(shared_num.at[j], tmp_v)
                num_tot = num_tot + tmp_v[...]
                pltpu.sync_copy(shared_den.at[j], tmp_v)
                den_tot = den_tot + tmp_v[...]
            acc_v[...] = num_tot
            pltpu.sync_copy(acc_v, num_out.at[c])
            acc_v[...] = den_tot
            pltpu.sync_copy(acc_v, den_out.at[c])

    return k


def _combine_body(num_ref, den_ref, out_ref):
    num = jnp.sum(num_ref[...], axis=0, keepdims=True)
    den = jnp.sum(den_ref[...], axis=0, keepdims=True) + 1.0
    out_ref[...] = jnp.full((1, L), jnp.sum(num / den), jnp.float32)


def kernel(prob, edge_index, w_p, w_n):
    n, kk = prob.shape
    e = w_p.shape[0]

    rpt = ((-(-n // NW)) + 7) // 8 * 8
    n_pad = rpt * NW
    prob_pad = jnp.zeros((n_pad, L), jnp.float32).at[:n, :kk].set(prob)

    nchunk = -(-e // (NW * CH))
    e_pad = nchunk * CH * NW
    pad = e_pad - e
    row = edge_index[0]
    col = edge_index[1]
    wp = w_p
    wn = w_n
    if pad:
        row = jnp.concatenate([row, jnp.zeros((pad,), jnp.int32)])
        col = jnp.concatenate([col, jnp.zeros((pad,), jnp.int32)])
        wp = jnp.concatenate([wp, jnp.zeros((pad,), jnp.float32)])
        wn = jnp.concatenate([wn, jnp.zeros((pad,), jnp.float32)])

    num_parts, den_parts = _sc_edge_kernel(nchunk, rpt)(
        prob_pad, row, col, wp, wn)

    out = pl.pallas_call(
        _combine_body,
        out_shape=jax.ShapeDtypeStruct((1, L), jnp.float32),
    )(num_parts, den_parts)
    return out[0, 0:1]
